# baseline (speedup 1.0000x reference)
import functools

import jax
import jax.numpy as jnp
from jax import lax
from jax.experimental import pallas as pl
from jax.experimental.pallas import tpu as pltpu
from jax.experimental.pallas import tpu_sc as plsc

SEQ_NUM = 1000000
N_EMBD = 64
WIN_LEN = 200
BATCH = 4096
INPUT_DIM = 64

_info = plsc.get_sparse_core_info()
_NC, _NS = _info.num_cores, _info.num_subcores
_NW = _NC * _NS
_B_PER_W = BATCH // _NW


def _sc_gather(idx, table):
    mesh = plsc.VectorSubcoreMesh(core_axis_name="c", subcore_axis_name="s")

    @functools.partial(
        pl.kernel,
        mesh=mesh,
        out_type=jax.ShapeDtypeStruct((BATCH, N_EMBD), jnp.float32),
        scratch_types=[
            pltpu.VMEM((_B_PER_W,), jnp.int32),
            pltpu.VMEM((_B_PER_W, N_EMBD), jnp.float32),
            pltpu.SemaphoreType.DMA,
        ],
        compiler_params=pltpu.CompilerParams(use_tc_tiling_on_sc=False),
    )
    def k(idx_hbm, table_hbm, out_hbm, idx_v, rows_v, sem):
        wid = lax.axis_index("s") * _NC + lax.axis_index("c")
        base = wid * _B_PER_W
        pltpu.sync_copy(idx_hbm.at[pl.ds(base, _B_PER_W)], idx_v)
        pltpu.async_copy(table_hbm.at[idx_v], rows_v, sem).wait()
        pltpu.sync_copy(rows_v, out_hbm.at[pl.ds(base, _B_PER_W)])

    return k(idx, table)


_BB = 64


def _tc_body(x_ref, id_ref, po_ref, o_ref):
    o_ref[:, :, 0:INPUT_DIM] = x_ref[...]
    o_ref[:, :, INPUT_DIM:] = id_ref[...][:, None, :] + po_ref[...][None, :, :]


def _tc_concat(x, id_emb, po_table):
    return pl.pallas_call(
        _tc_body,
        grid=(BATCH // _BB,),
        in_specs=[
            pl.BlockSpec((_BB, WIN_LEN, INPUT_DIM), lambda i: (i, 0, 0)),
            pl.BlockSpec((_BB, N_EMBD), lambda i: (i, 0)),
            pl.BlockSpec((WIN_LEN, N_EMBD), lambda i: (0, 0)),
        ],
        out_specs=pl.BlockSpec((_BB, WIN_LEN, INPUT_DIM + N_EMBD), lambda i: (i, 0, 0)),
        out_shape=jax.ShapeDtypeStruct((BATCH, WIN_LEN, INPUT_DIM + N_EMBD), jnp.float32),
    )(x, id_emb, po_table)


@jax.jit
def kernel(series_id, x, id_table, po_table):
    id_emb = _sc_gather(series_id.astype(jnp.int32), id_table)
    return _tc_concat(x, id_emb, po_table)
